# baseline (speedup 1.0000x reference)
import jax
import jax.numpy as jnp
from jax import lax
from jax.experimental import pallas as pl
from jax.experimental.pallas import tpu as pltpu
from jax.experimental.pallas import tpu_sc as plsc

VOCAB = 100000
D = 32
NC, NS = 2, 16
NW = NC * NS
BATCH = 16384
HIST = 50
BW = BATCH // NW
NBT = BW // 128
NDT = D // 8
NSTEP = HIST * 4
TBLK = NDT * 8 * 128


def _body(xt_ref, w0, w1, w2, w3, o0, o1, o2, o3, xv, gbuf, tbuf, cvec, gsem, ssem):
    wid = lax.axis_index("s") * NC + lax.axis_index("c")
    tables = (w0, w1, w2, w3)
    outs = (o0, o1, o2, o3)
    b0 = wid * BW
    iota = lax.iota(jnp.int32, 16)

    for j in range(16):
        rot = (iota + j) & 15
        cvec[0, j, :] = rot
        cvec[1, j, :] = ((rot >> 3) << 12) + ((rot & 7) << 7) + iota

    def fire_gather(h, t, b):
        for k in range(4):
            @pl.when(t == k)
            def _():
                pltpu.async_copy(
                    tables[k].at[xv.at[h]], gbuf.at[pl.ds(b * BW, BW)], gsem
                )

    def wait_gather():
        pltpu.make_async_copy(
            w0.at[pl.ds(0, BW)], gbuf.at[pl.ds(0, BW)], gsem
        ).wait()

    def fire_stores(h, t, b):
        for k in range(4):
            @pl.when(t == k)
            def _():
                for dt in range(NDT):
                    pltpu.async_copy(
                        tbuf.at[pl.ds(b * 4 * TBLK + dt * TBLK, TBLK)],
                        outs[k].at[h].at[dt].at[pl.ds(wid * TBLK, TBLK)],
                        ssem,
                    )

    def wait_stores():
        for dt in range(NDT):
            pltpu.make_async_copy(
                tbuf.at[pl.ds(dt * TBLK, TBLK)],
                o0.at[0].at[0].at[pl.ds(0, TBLK)],
                ssem,
            ).wait()

    def transpose(b):
        for c in range(2):

            def m_body(m, carry):
                bt = m // 8
                kb = lax.rem(m, 8)
                rowvec = (b * BW + bt * 128 + kb * 16) + iota
                baseo = b * 16384 + c * 8192 + bt * 1024 + kb * 16
                for j in range(16):
                    col = cvec[0, j, :] + (c * 16)
                    pos = cvec[1, j, :] + baseo
                    v = plsc.load_gather(gbuf, [rowvec, col])
                    plsc.store_scatter(tbuf, [pos], v)
                return carry

            lax.fori_loop(0, NBT * 8, m_body, 0)

    pltpu.sync_copy(xt_ref.at[:, pl.ds(b0, BW)], xv)
    fire_gather(0, 0, 0)

    def step(i, carry):
        h = i // 4
        t = lax.rem(i, 4)
        b = lax.rem(i, 2)
        ni = i + 1

        @pl.when(ni < NSTEP)
        def _():
            fire_gather(ni // 4, lax.rem(ni, 4), lax.rem(ni, 2))

        wait_gather()

        @pl.when(i >= 2)
        def _():
            wait_stores()

        transpose(b)
        fire_stores(h, t, b)
        return carry

    lax.fori_loop(0, NSTEP, step, 0)
    wait_stores()
    wait_stores()


@jax.jit
def _sc_gather(xt, W0, W1, W2, W3):
    f = pl.kernel(
        _body,
        out_type=[
            jax.ShapeDtypeStruct((HIST, NDT, (BATCH // 128) * 8 * 128), jnp.float32)
        ] * 4,
        mesh=plsc.VectorSubcoreMesh(core_axis_name="c", subcore_axis_name="s"),
        scratch_types=[
            pltpu.VMEM((HIST, BW), jnp.int32),
            pltpu.VMEM((2 * BW, D), jnp.float32),
            pltpu.VMEM((2 * 4 * TBLK,), jnp.float32),
            pltpu.VMEM((2, 16, 16), jnp.int32),
            pltpu.SemaphoreType.DMA,
            pltpu.SemaphoreType.DMA,
        ],
        compiler_params=pltpu.CompilerParams(
            use_tc_tiling_on_sc=False, needs_layout_passes=False
        ),
    )
    return f(xt, W0, W1, W2, W3)


def kernel(x, W0, W1, W2, W3):
    xt = x.T.astype(jnp.int32)
    outs = _sc_gather(xt, W0, W1, W2, W3)
    return tuple(
        o.reshape(HIST, NDT, BATCH // 128, 8, 128)
        .transpose(2, 4, 0, 1, 3)
        .reshape(BATCH, HIST, D)
        for o in outs
    )


# speedup vs baseline: 1.0016x; 1.0016x over previous
import jax
import jax.numpy as jnp
from jax import lax
from jax.experimental import pallas as pl
from jax.experimental.pallas import tpu as pltpu
from jax.experimental.pallas import tpu_sc as plsc

VOCAB = 100000
D = 32
NC, NS = 2, 16
NW = NC * NS
BATCH = 16384
HIST = 50
BW = BATCH // NW
NBT = BW // 128
NDT = D // 8
NSTEP = HIST * 4
TBLK = NDT * 8 * 128


def _body(xt_ref, w0, w1, w2, w3, o0, o1, o2, o3, xv, gbuf, tbuf, cvec, gsem, ssem):
    wid = lax.axis_index("s") * NC + lax.axis_index("c")
    tables = (w0, w1, w2, w3)
    outs = (o0, o1, o2, o3)
    b0 = wid * BW
    iota = lax.iota(jnp.int32, 16)

    for j in range(16):
        rot = (iota + j) & 15
        cvec[0, j, :] = rot
        cvec[1, j, :] = rot + 16
        cvec[2, j, :] = ((rot >> 3) << 12) + ((rot & 7) << 7) + iota

    def fire_gather(h, t, b):
        for k in range(4):
            @pl.when(t == k)
            def _():
                pltpu.async_copy(
                    tables[k].at[xv.at[h]], gbuf.at[pl.ds(b * BW, BW)], gsem
                )

    def wait_gather():
        pltpu.make_async_copy(
            w0.at[pl.ds(0, BW)], gbuf.at[pl.ds(0, BW)], gsem
        ).wait()

    def fire_stores(h, t, b):
        for k in range(4):
            @pl.when(t == k)
            def _():
                for dt in range(NDT):
                    pltpu.async_copy(
                        tbuf.at[pl.ds(b * 4 * TBLK + dt * TBLK, TBLK)],
                        outs[k].at[h].at[dt].at[pl.ds(wid * TBLK, TBLK)],
                        ssem,
                    )

    def wait_stores():
        for dt in range(NDT):
            pltpu.make_async_copy(
                tbuf.at[pl.ds(dt * TBLK, TBLK)],
                o0.at[0].at[0].at[pl.ds(0, TBLK)],
                ssem,
            ).wait()

    def transpose(b):
        for c in range(2):

            def m_body(m, carry):
                bt = m // 8
                kb = lax.rem(m, 8)
                rowvec = (b * BW + bt * 128 + kb * 16) + iota
                baseo = b * 16384 + c * 8192 + bt * 1024 + kb * 16
                for j in range(16):
                    col = cvec[c, j, :]
                    pos = cvec[2, j, :] + baseo
                    v = plsc.load_gather(gbuf, [rowvec, col])
                    plsc.store_scatter(tbuf, [pos], v)
                return carry

            lax.fori_loop(0, NBT * 8, m_body, 0)

    pltpu.sync_copy(xt_ref.at[:, pl.ds(b0, BW)], xv)
    fire_gather(0, 0, 0)

    def step(i, carry):
        h = i // 4
        t = lax.rem(i, 4)
        b = lax.rem(i, 2)
        ni = i + 1

        @pl.when(ni < NSTEP)
        def _():
            fire_gather(ni // 4, lax.rem(ni, 4), lax.rem(ni, 2))

        wait_gather()

        @pl.when(i >= 2)
        def _():
            wait_stores()

        transpose(b)
        fire_stores(h, t, b)
        return carry

    lax.fori_loop(0, NSTEP, step, 0)
    wait_stores()
    wait_stores()


@jax.jit
def _sc_gather(xt, W0, W1, W2, W3):
    f = pl.kernel(
        _body,
        out_type=[
            jax.ShapeDtypeStruct((HIST, NDT, (BATCH // 128) * 8 * 128), jnp.float32)
        ] * 4,
        mesh=plsc.VectorSubcoreMesh(core_axis_name="c", subcore_axis_name="s"),
        scratch_types=[
            pltpu.VMEM((HIST, BW), jnp.int32),
            pltpu.VMEM((2 * BW, D), jnp.float32),
            pltpu.VMEM((2 * 4 * TBLK,), jnp.float32),
            pltpu.VMEM((3, 16, 16), jnp.int32),
            pltpu.SemaphoreType.DMA,
            pltpu.SemaphoreType.DMA,
        ],
        compiler_params=pltpu.CompilerParams(
            use_tc_tiling_on_sc=False, needs_layout_passes=False
        ),
    )
    return f(xt, W0, W1, W2, W3)


def kernel(x, W0, W1, W2, W3):
    xt = x.T.astype(jnp.int32)
    outs = _sc_gather(xt, W0, W1, W2, W3)
    return tuple(
        o.reshape(HIST, NDT, BATCH // 128, 8, 128)
        .transpose(2, 4, 0, 1, 3)
        .reshape(BATCH, HIST, D)
        for o in outs
    )
